# Initial kernel scaffold; baseline (speedup 1.0000x reference)
import jax
import jax.numpy as jnp
from jax.experimental import pallas as pl


def kernel(x, edge_index, premise_index, batch, params):
    raise NotImplementedError("write your pallas kernel here")


# speedup vs baseline: 24.4384x; 24.4384x over previous
import functools

import jax
import jax.numpy as jnp
from jax import lax
from jax.experimental import pallas as pl
from jax.experimental.pallas import tpu as pltpu
from jax.experimental.pallas import tpu_sc as plsc

N = 10000
NPAD = 10016
E = 640000
K = 8
C_ALL = 400
LAYERS = 24
VOCAB = 2000

CHUNK = 128
NCORES = 2
NSUB = 16
NW = NCORES * NSUB
CPW = 160
EPAD = NW * CPW * CHUNK
ROWS_PER_SUB = NPAD // NSUB
RING = 16
AHEAD = 8

EMB_CPW = 4
EMB_CHUNK = 80
PREM_CHUNK = 32

_SC_PARAMS = dict(use_tc_tiling_on_sc=False)


@functools.cache
def _sc_kernels():
  mesh = plsc.VectorSubcoreMesh(core_axis_name="c", subcore_axis_name="s",
                                num_cores=NCORES, num_subcores=NSUB)

  @functools.partial(
      pl.kernel,
      out_type=(
          jax.ShapeDtypeStruct((NCORES, NPAD, K), jnp.float32),
          jax.ShapeDtypeStruct((NCORES, NPAD, K), jnp.float32),
      ),
      mesh=mesh,
      compiler_params=pltpu.CompilerParams(**_SC_PARAMS),
      scratch_types=(
          pltpu.VMEM((CPW, CHUNK), jnp.int32),
          pltpu.VMEM((CPW, CHUNK), jnp.int32),
          pltpu.VMEM((RING, CHUNK, K), jnp.float32),
          pltpu.VMEM((RING, CHUNK, K), jnp.float32),
          pltpu.VMEM_SHARED((NPAD, K), jnp.float32),
          pltpu.VMEM_SHARED((NPAD, K), jnp.float32),
          pltpu.SemaphoreType.DMA,
          pltpu.SemaphoreType.DMA,
          pltpu.SemaphoreType.DMA,
          pltpu.SemaphoreType.DMA,
      ),
  )
  def edge_pass(gin_hbm, gout_hbm, srcp_hbm, dstp_hbm, pin_hbm, pout_hbm,
                idx_s, idx_d, ring_in, ring_out, acc_in, acc_out,
                sem_gi, sem_go, sem_si, sem_so):
    c = lax.axis_index("c")
    s = lax.axis_index("s")
    w = c * NSUB + s
    r0 = s * ROWS_PER_SUB
    pltpu.sync_copy(gin_hbm.at[pl.ds(r0, ROWS_PER_SUB)],
                    acc_in.at[pl.ds(r0, ROWS_PER_SUB)])
    pltpu.sync_copy(gout_hbm.at[pl.ds(r0, ROWS_PER_SUB)],
                    acc_out.at[pl.ds(r0, ROWS_PER_SUB)])
    pltpu.sync_copy(srcp_hbm.at[pl.ds(w * CPW, CPW)], idx_s)
    pltpu.sync_copy(dstp_hbm.at[pl.ds(w * CPW, CPW)], idx_d)
    plsc.subcore_barrier()

    def fire(j, b):
      pltpu.async_copy(gin_hbm.at[idx_s.at[j]], ring_in.at[b], sem_gi)
      pltpu.async_copy(gout_hbm.at[idx_d.at[j]], ring_out.at[b], sem_go)

    for b in range(AHEAD):
      fire(b, b)

    def superstep(t, carry):
      for b in range(RING):
        j = t * RING + b
        bn = (b + AHEAD) % RING
        pltpu.make_async_copy(gin_hbm.at[idx_s.at[j]], ring_in.at[b],
                              sem_gi).wait()
        pltpu.async_copy(ring_in.at[b], acc_in.at[idx_d.at[j]], sem_si,
                         add=True)
        pltpu.make_async_copy(gout_hbm.at[idx_d.at[j]], ring_out.at[b],
                              sem_go).wait()
        pltpu.async_copy(ring_out.at[b], acc_out.at[idx_s.at[j]], sem_so,
                         add=True)
        jp = j - AHEAD

        @pl.when(jp >= 0)
        def _():
          pltpu.make_async_copy(ring_in.at[bn], acc_in.at[idx_d.at[jp]],
                                sem_si).wait()
          pltpu.make_async_copy(ring_out.at[bn], acc_out.at[idx_s.at[jp]],
                                sem_so).wait()

        jn = j + AHEAD

        @pl.when(jn < CPW)
        def _():
          fire(jn, bn)
      return carry

    lax.fori_loop(0, CPW // RING, superstep, 0)

    for b in range(AHEAD):
      jp = CPW - AHEAD + b
      pltpu.make_async_copy(ring_in.at[AHEAD + b], acc_in.at[idx_d.at[jp]],
                            sem_si).wait()
      pltpu.make_async_copy(ring_out.at[AHEAD + b], acc_out.at[idx_s.at[jp]],
                            sem_so).wait()
    plsc.subcore_barrier()
    pltpu.sync_copy(acc_in.at[pl.ds(r0, ROWS_PER_SUB)],
                    pin_hbm.at[c, pl.ds(r0, ROWS_PER_SUB)])
    pltpu.sync_copy(acc_out.at[pl.ds(r0, ROWS_PER_SUB)],
                    pout_hbm.at[c, pl.ds(r0, ROWS_PER_SUB)])

  @functools.partial(
      pl.kernel,
      out_type=jax.ShapeDtypeStruct((NW * EMB_CPW * EMB_CHUNK, 2 * K),
                                    jnp.float32),
      mesh=mesh,
      compiler_params=pltpu.CompilerParams(**_SC_PARAMS),
      scratch_types=(
          pltpu.VMEM((EMB_CPW, EMB_CHUNK), jnp.int32),
          pltpu.VMEM((EMB_CHUNK, 2 * K), jnp.float32),
          pltpu.SemaphoreType.DMA,
      ),
  )
  def emb_gather(table_hbm, idxp_hbm, out_hbm, idx_v, buf, sem):
    c = lax.axis_index("c")
    s = lax.axis_index("s")
    w = c * NSUB + s
    pltpu.sync_copy(idxp_hbm.at[pl.ds(w * EMB_CPW, EMB_CPW)], idx_v)
    for j in range(EMB_CPW):
      pltpu.async_copy(table_hbm.at[idx_v.at[j]], buf, sem).wait()
      pltpu.sync_copy(buf, out_hbm.at[pl.ds((w * EMB_CPW + j) * EMB_CHUNK,
                                            EMB_CHUNK)])

  @functools.partial(
      pl.kernel,
      out_type=jax.ShapeDtypeStruct((NW * PREM_CHUNK, C_ALL), jnp.float32),
      mesh=mesh,
      compiler_params=pltpu.CompilerParams(**_SC_PARAMS),
      scratch_types=(
          pltpu.VMEM((1, PREM_CHUNK), jnp.int32),
          pltpu.VMEM((PREM_CHUNK, C_ALL), jnp.float32),
          pltpu.SemaphoreType.DMA,
      ),
  )
  def premise_gather(buf_hbm, idxp_hbm, out_hbm, idx_v, buf, sem):
    c = lax.axis_index("c")
    s = lax.axis_index("s")
    w = c * NSUB + s
    pltpu.sync_copy(idxp_hbm.at[pl.ds(w, 1)], idx_v)
    pltpu.async_copy(buf_hbm.at[idx_v.at[0]], buf, sem).wait()
    pltpu.sync_copy(buf, out_hbm.at[pl.ds(w * PREM_CHUNK, PREM_CHUNK)])

  return edge_pass, emb_gather, premise_gather


def _mm_vpu(a, b):
    cols = [jnp.sum(a * b[:, j].reshape(1, -1), axis=1, keepdims=True)
            for j in range(b.shape[1])]
    return jnp.concatenate(cols, axis=1)


def _bn_cols(h, gamma, beta):
    mu = jnp.mean(h, axis=0, keepdims=True)
    d = h - mu
    var = jnp.mean(d * d, axis=0, keepdims=True)
    return gamma * d * lax.rsqrt(var + 1e-5) + beta


def _convfin_body(pin, pout, gin, gout, dvi, dvo, Wi, bi, Wo, bo, newcols):
    def emit(i, carry):
        r = pl.ds(i * _RB, _RB)
        sin = (pin[0, r, :] + pin[1, r, :] - gin[r, :]) * dvi[r, :]
        sout = (pout[0, r, :] + pout[1, r, :] - gout[r, :]) * dvo[r, :]
        in_x = _mm_vpu(sin, Wi[:, :]) + bi[:, :]
        out_x = _mm_vpu(sout, Wo[:, :]) + bo[:, :]
        newcols[r, :] = jnp.concatenate([in_x, out_x], axis=1)
        return carry

    lax.fori_loop(0, _NB, emit, 0)


_convfin = pl.pallas_call(
    _convfin_body,
    out_shape=jax.ShapeDtypeStruct((N, 2 * K), jnp.float32),
)


_NB = 10
_RB = N // _NB


def _dense_body(buf, fg, fb, Wf, bf, cg, cb, dvi, dvo, gin_new, gout_new):
    def stats1(i, s):
        h = jnp.maximum(buf[pl.ds(i * _RB, _RB), :], 0.0)
        return s + jnp.sum(h, axis=0, keepdims=True)

    s = lax.fori_loop(0, _NB, stats1, jnp.zeros((1, C_ALL), jnp.float32))
    mu = s * (1.0 / N)

    def stats1b(i, s2):
        h = jnp.maximum(buf[pl.ds(i * _RB, _RB), :], 0.0)
        d = h - mu
        return s2 + jnp.sum(d * d, axis=0, keepdims=True)

    s2 = lax.fori_loop(0, _NB, stats1b, jnp.zeros((1, C_ALL), jnp.float32))
    var = s2 * (1.0 / N)
    rs = lax.rsqrt(var + 1e-5)
    scale = fg[:, :] * rs
    shift = fb[:, :] - mu * scale

    def fc_block(i):
        h = jnp.maximum(buf[pl.ds(i * _RB, _RB), :], 0.0)
        hn = h * scale + shift
        hb = _mm_vpu(hn, Wf[:, :]) + bf[:, :]
        return jnp.maximum(hb, 0.0)

    def stats2(i, carry):
        sh, sh2 = carry
        hb = fc_block(i)
        return (sh + jnp.sum(hb, axis=0, keepdims=True),
                sh2 + jnp.sum(hb * hb, axis=0, keepdims=True))

    sh, sh2 = lax.fori_loop(0, _NB, stats2,
                            (jnp.zeros((1, K), jnp.float32),
                             jnp.zeros((1, K), jnp.float32)))
    muh = sh * (1.0 / N)
    varh = sh2 * (1.0 / N) - muh * muh
    rsh = lax.rsqrt(varh + 1e-5)
    scale2 = cg[:, :] * rsh
    shift2 = cb[:, :] - muh * scale2

    def emit(i, carry):
        hn = fc_block(i) * scale2 + shift2
        gin_new[pl.ds(i * _RB, _RB), :] = hn * dvi[pl.ds(i * _RB, _RB), :]
        gout_new[pl.ds(i * _RB, _RB), :] = hn * dvo[pl.ds(i * _RB, _RB), :]
        return carry

    lax.fori_loop(0, _NB, emit, 0)
    z = jnp.zeros((NPAD - N, K), jnp.float32)
    gin_new[N:, :] = z
    gout_new[N:, :] = z


_dense = pl.pallas_call(
    _dense_body,
    out_shape=(
        jax.ShapeDtypeStruct((NPAD, K), jnp.float32),
        jax.ShapeDtypeStruct((NPAD, K), jnp.float32),
    ),
)


def _head_body(prem, og, ob, Wo, bo, out):
    h = jnp.maximum(prem[:, :], 0.0)
    h = _bn_cols(h, og[:, :], ob[:, :])
    out[:, :] = _mm_vpu(h, Wo[:, :]) + bo[:, :]


_head = pl.pallas_call(
    _head_body,
    out_shape=jax.ShapeDtypeStruct((NW * PREM_CHUNK, 1), jnp.float32),
)


def kernel(x, edge_index, premise_index, batch, params):
    del batch
    _edge_pass, _emb_gather, _premise_gather = _sc_kernels()
    src = edge_index[0]
    dst = edge_index[1]
    f32 = jnp.float32

    ones_e = jnp.ones((E,), f32)
    deg_in = jax.ops.segment_sum(ones_e, dst, num_segments=N) + 1.0
    deg_out = jax.ops.segment_sum(ones_e, src, num_segments=N) + 1.0
    dinv_in = lax.rsqrt(deg_in)
    dinv_out = lax.rsqrt(deg_out)
    zpad = jnp.zeros((NPAD - N,), f32)
    dvi = jnp.concatenate([dinv_in, zpad]).reshape(NPAD, 1)
    dvo = jnp.concatenate([dinv_out, zpad]).reshape(NPAD, 1)

    epad_idx = jnp.full((EPAD - E,), N, jnp.int32)
    srcp = jnp.concatenate([src, epad_idx]).reshape(NW * CPW, CHUNK)
    dstp = jnp.concatenate([dst, epad_idx]).reshape(NW * CPW, CHUNK)

    xe = jnp.concatenate([x, jnp.zeros((NW * EMB_CPW * EMB_CHUNK - N,),
                                       jnp.int32)]).reshape(NW * EMB_CPW,
                                                            EMB_CHUNK)
    emb_rows = _emb_gather(params["emb"], xe)[:N]

    buf = jnp.zeros((N, C_ALL), f32)
    buf = lax.dynamic_update_slice(buf, emb_rows, (0, 0))

    def pad_cols(v, fill):
        return jnp.concatenate(
            [v, jnp.full((C_ALL - v.shape[0],), fill, f32)]).reshape(1, C_ALL)

    def pad_W(W):
        return jnp.concatenate(
            [W, jnp.zeros((C_ALL - W.shape[0], K), f32)], axis=0)

    gin = gout = pin = pout = None
    for l, lp in enumerate(params["layers"]):
        if l > 0:
            lq = params["layers"][l - 1]
            newcols = _convfin(pin, pout, gin, gout, dvi, dvo,
                               lq["in_W"], lq["in_b"].reshape(1, K),
                               lq["out_W"], lq["out_b"].reshape(1, K))
            buf = lax.dynamic_update_slice(buf, newcols, (0, 2 * K * l))
        gin, gout = _dense(buf,
                           pad_cols(lp["fc_gamma"], 1.0),
                           pad_cols(lp["fc_beta"], 0.0),
                           pad_W(lp["fc_W"]),
                           lp["fc_b"].reshape(1, K),
                           lp["cv_gamma"].reshape(1, K),
                           lp["cv_beta"].reshape(1, K),
                           dvi, dvo)
        pin, pout = _edge_pass(gin, gout, srcp, dstp)

    lq = params["layers"][LAYERS - 1]
    newcols = _convfin(pin, pout, gin, gout, dvi, dvo,
                       lq["in_W"], lq["in_b"].reshape(1, K),
                       lq["out_W"], lq["out_b"].reshape(1, K))
    buf = lax.dynamic_update_slice(buf, newcols, (0, 2 * K * LAYERS))

    prem = _premise_gather(buf, premise_index.reshape(NW, PREM_CHUNK))
    out = _head(prem,
                params["out_gamma"].reshape(1, C_ALL),
                params["out_beta"].reshape(1, C_ALL),
                params["out_W"],
                params["out_b"].reshape(1, 1))
    return out.reshape(NW * PREM_CHUNK)
